# initial kernel scaffold (unmeasured)
import jax
import jax.numpy as jnp
from jax import lax
from jax.experimental import pallas as pl
from jax.experimental.pallas import tpu as pltpu

N_ROUNDS = 3


def kernel(x, dy, gamma):
    del gamma
    m_per, d = x.shape
    qr = m_per // 4

    def body(x_hbm, dy_hbm, out_ref,
             xq, dyq, acc, rbuf, copy_sems, send_sems, recv_sems):
        mx = lax.axis_index("x")
        my = lax.axis_index("y")
        mz = lax.axis_index("z")
        q = mx * 2 + mz
        row0 = q * qr

        cp_x = pltpu.make_async_copy(
            x_hbm.at[pl.ds(row0, qr)], xq, copy_sems.at[0])
        cp_dy = pltpu.make_async_copy(
            dy_hbm.at[pl.ds(row0, qr)], dyq, copy_sems.at[1])
        cp_x.start()
        cp_dy.start()

        partners = [(1 - mx, my, mz), (mx, 1 - my, mz), (mx, my, 1 - mz)]
        barrier = pltpu.get_barrier_semaphore()
        for pid in partners:
            pl.semaphore_signal(
                barrier, inc=1, device_id=pid,
                device_id_type=pl.DeviceIdType.MESH)
        pl.semaphore_wait(barrier, N_ROUNDS)

        cp_x.wait()
        cp_dy.wait()

        xv = xq[:, :]
        dyv = dyq[:, :]
        mu = jnp.mean(xv, axis=1, keepdims=True)
        xc = xv - mu
        var = jnp.mean(xc * xc, axis=1, keepdims=True)
        xhat = xc * lax.rsqrt(var + 1e-5)
        acc[0, :] = jnp.sum(dyv * xhat, axis=0)
        acc[1, :] = jnp.sum(dyv, axis=0)

        for r, pid in enumerate(partners):
            rdma = pltpu.make_async_remote_copy(
                src_ref=acc,
                dst_ref=rbuf.at[r],
                send_sem=send_sems.at[r],
                recv_sem=recv_sems.at[r],
                device_id=pid,
                device_id_type=pl.DeviceIdType.MESH,
            )
            rdma.start()
            rdma.wait()
            acc[:, :] = acc[:, :] + rbuf[r]

        out_ref[:, :] = acc[:, :]

    return pl.pallas_call(
        body,
        out_shape=jax.ShapeDtypeStruct((2, d), jnp.float32),
        in_specs=[
            pl.BlockSpec(memory_space=pltpu.ANY),
            pl.BlockSpec(memory_space=pltpu.ANY),
        ],
        out_specs=pl.BlockSpec(memory_space=pltpu.VMEM),
        scratch_shapes=[
            pltpu.VMEM((qr, d), jnp.float32),
            pltpu.VMEM((qr, d), jnp.float32),
            pltpu.VMEM((2, d), jnp.float32),
            pltpu.VMEM((N_ROUNDS, 2, d), jnp.float32),
            pltpu.SemaphoreType.DMA((2,)),
            pltpu.SemaphoreType.DMA((N_ROUNDS,)),
            pltpu.SemaphoreType.DMA((N_ROUNDS,)),
        ],
        compiler_params=pltpu.CompilerParams(collective_id=0),
    )(x, dy)


# baseline (device time: 18210 ns/iter reference)
import jax
import jax.numpy as jnp
from jax import lax
from jax.experimental import pallas as pl
from jax.experimental.pallas import tpu as pltpu

N_ROUNDS = 3


def kernel(x, dy, gamma):
    del gamma
    m_per, d = x.shape
    qr = m_per // 4

    def body(x_hbm, dy_hbm, out_ref,
             xq, dyq, acc, rbuf, copy_sems, send_sems, recv_sems):
        mx = lax.axis_index("x")
        my = lax.axis_index("y")
        mz = lax.axis_index("z")
        q = mx * 2 + mz
        row0 = q * qr

        cp_x = pltpu.make_async_copy(
            x_hbm.at[pl.ds(row0, qr)], xq, copy_sems.at[0])
        cp_dy = pltpu.make_async_copy(
            dy_hbm.at[pl.ds(row0, qr)], dyq, copy_sems.at[1])
        cp_x.start()
        cp_dy.start()

        partners = [(1 - mx, my, mz), (mx, 1 - my, mz), (mx, my, 1 - mz)]
        barrier = pltpu.get_barrier_semaphore()
        for pid in partners:
            pl.semaphore_signal(
                barrier, inc=1, device_id=pid,
                device_id_type=pl.DeviceIdType.MESH)
        pl.semaphore_wait(barrier, N_ROUNDS)

        cp_x.wait()
        cp_dy.wait()

        xv = xq[:, :]
        dyv = dyq[:, :]
        mu = jnp.mean(xv, axis=1, keepdims=True)
        xc = xv - mu
        var = jnp.mean(xc * xc, axis=1, keepdims=True)
        xhat = xc * lax.rsqrt(var + 1e-5)
        acc[0, :] = jnp.sum(dyv * xhat, axis=0)
        acc[1, :] = jnp.sum(dyv, axis=0)

        for r, pid in enumerate(partners):
            rdma = pltpu.make_async_remote_copy(
                src_ref=acc,
                dst_ref=rbuf.at[r],
                send_sem=send_sems.at[r],
                recv_sem=recv_sems.at[r],
                device_id=pid,
                device_id_type=pl.DeviceIdType.MESH,
            )
            rdma.start()
            rdma.wait()
            acc[:, :] = acc[:, :] + rbuf[r]

        out_ref[:, :] = acc[:, :]

    return pl.pallas_call(
        body,
        out_shape=jax.ShapeDtypeStruct((2, d), jnp.float32),
        in_specs=[
            pl.BlockSpec(memory_space=pl.ANY),
            pl.BlockSpec(memory_space=pl.ANY),
        ],
        out_specs=pl.BlockSpec(memory_space=pltpu.VMEM),
        scratch_shapes=[
            pltpu.VMEM((qr, d), jnp.float32),
            pltpu.VMEM((qr, d), jnp.float32),
            pltpu.VMEM((2, d), jnp.float32),
            pltpu.VMEM((N_ROUNDS, 2, d), jnp.float32),
            pltpu.SemaphoreType.DMA((2,)),
            pltpu.SemaphoreType.DMA((N_ROUNDS,)),
            pltpu.SemaphoreType.DMA((N_ROUNDS,)),
        ],
        compiler_params=pltpu.CompilerParams(collective_id=0),
    )(x, dy)


# device time: 14017 ns/iter; 1.2991x vs baseline; 1.2991x over previous
import jax
import jax.numpy as jnp
from jax import lax
from jax.experimental import pallas as pl
from jax.experimental.pallas import tpu as pltpu

N_DEV = 8
N_CHUNKS = 4


def kernel(x, dy, gamma):
    del gamma
    m_per, d = x.shape
    qr = m_per // 4
    cr = qr // N_CHUNKS

    def body(x_hbm, dy_hbm, out_ref,
             xq, dyq, acc, rbuf, x_sems, dy_sems, send_sems, recv_sems):
        mx = lax.axis_index("x")
        my = lax.axis_index("y")
        mz = lax.axis_index("z")
        row0 = (mx * 2 + mz) * qr

        x_cps, dy_cps = [], []
        for c in range(N_CHUNKS):
            cp = pltpu.make_async_copy(
                x_hbm.at[pl.ds(row0 + c * cr, cr)],
                xq.at[pl.ds(c * cr, cr)], x_sems.at[c])
            cp.start()
            x_cps.append(cp)
            cp = pltpu.make_async_copy(
                dy_hbm.at[pl.ds(row0 + c * cr, cr)],
                dyq.at[pl.ds(c * cr, cr)], dy_sems.at[c])
            cp.start()
            dy_cps.append(cp)

        peers = []
        for a in range(2):
            for b in range(2):
                for c in range(2):
                    if a == b == c == 0:
                        continue
                    peers.append((a * 4 + b * 2 + c,
                                  (a + mx - 2 * a * mx,
                                   b + my - 2 * b * my,
                                   c + mz - 2 * c * mz)))
        barrier = pltpu.get_barrier_semaphore()
        for _, pid in peers:
            pl.semaphore_signal(
                barrier, inc=1, device_id=pid,
                device_id_type=pl.DeviceIdType.MESH)
        pl.semaphore_wait(barrier, N_DEV - 1)

        for c in range(N_CHUNKS):
            x_cps[c].wait()
            dy_cps[c].wait()
            xv = xq[pl.ds(c * cr, cr), :]
            dyv = dyq[pl.ds(c * cr, cr), :]
            mu = jnp.mean(xv, axis=1, keepdims=True)
            xc = xv - mu
            var = jnp.mean(xc * xc, axis=1, keepdims=True)
            xhat = xc * lax.rsqrt(var + 1e-5)
            dg = jnp.sum(dyv * xhat, axis=0)
            db = jnp.sum(dyv, axis=0)
            if c == 0:
                acc[0, :] = dg
                acc[1, :] = db
            else:
                acc[0, :] = acc[0, :] + dg
                acc[1, :] = acc[1, :] + db

        rdmas = []
        for slot, pid in peers:
            rdma = pltpu.make_async_remote_copy(
                src_ref=acc,
                dst_ref=rbuf.at[slot],
                send_sem=send_sems.at[slot],
                recv_sem=recv_sems.at[slot],
                device_id=pid,
                device_id_type=pl.DeviceIdType.MESH,
            )
            rdma.start()
            rdmas.append(rdma)
        for rdma in rdmas:
            rdma.wait_recv()
        total = acc[:, :]
        for slot, _ in peers:
            total = total + rbuf[slot]
        out_ref[:, :] = total
        for rdma in rdmas:
            rdma.wait_send()

    return pl.pallas_call(
        body,
        out_shape=jax.ShapeDtypeStruct((2, d), jnp.float32),
        in_specs=[
            pl.BlockSpec(memory_space=pl.ANY),
            pl.BlockSpec(memory_space=pl.ANY),
        ],
        out_specs=pl.BlockSpec(memory_space=pltpu.VMEM),
        scratch_shapes=[
            pltpu.VMEM((qr, d), jnp.float32),
            pltpu.VMEM((qr, d), jnp.float32),
            pltpu.VMEM((2, d), jnp.float32),
            pltpu.VMEM((N_DEV, 2, d), jnp.float32),
            pltpu.SemaphoreType.DMA((N_CHUNKS,)),
            pltpu.SemaphoreType.DMA((N_CHUNKS,)),
            pltpu.SemaphoreType.DMA((N_DEV,)),
            pltpu.SemaphoreType.DMA((N_DEV,)),
        ],
        compiler_params=pltpu.CompilerParams(collective_id=0),
    )(x, dy)


# device time: 13381 ns/iter; 1.3609x vs baseline; 1.0475x over previous
import jax
import jax.numpy as jnp
from jax import lax
from jax.experimental import pallas as pl
from jax.experimental.pallas import tpu as pltpu

N_DEV = 8
N_CHUNKS = 8
SPLIT = 6


def kernel(x, dy, gamma):
    del gamma
    m_per, d = x.shape
    qr = m_per // 4
    cr = qr // N_CHUNKS

    def body(x_hbm, dy_hbm, out_ref,
             xq, dyq, acc_a, acc_b, rbuf_a, rbuf_b,
             x_sems, dy_sems, send_a, recv_a, send_b, recv_b):
        mx = lax.axis_index("x")
        my = lax.axis_index("y")
        mz = lax.axis_index("z")
        row0 = (mx * 2 + mz) * qr

        x_cps, dy_cps = [], []
        for c in range(N_CHUNKS):
            cp = pltpu.make_async_copy(
                x_hbm.at[pl.ds(row0 + c * cr, cr)],
                xq.at[pl.ds(c * cr, cr)], x_sems.at[c])
            cp.start()
            x_cps.append(cp)
            cp = pltpu.make_async_copy(
                dy_hbm.at[pl.ds(row0 + c * cr, cr)],
                dyq.at[pl.ds(c * cr, cr)], dy_sems.at[c])
            cp.start()
            dy_cps.append(cp)

        peers = []
        for a in range(2):
            for b in range(2):
                for c in range(2):
                    if a == b == c == 0:
                        continue
                    peers.append((a * 4 + b * 2 + c, a + b + c,
                                  (a + mx - 2 * a * mx,
                                   b + my - 2 * b * my,
                                   c + mz - 2 * c * mz)))
        peers.sort(key=lambda p: -p[1])

        barrier = pltpu.get_barrier_semaphore()
        for _, _, pid in peers:
            pl.semaphore_signal(
                barrier, inc=1, device_id=pid,
                device_id_type=pl.DeviceIdType.MESH)

        def chunk_partial(c):
            x_cps[c].wait()
            dy_cps[c].wait()
            xv = xq[pl.ds(c * cr, cr), :]
            dyv = dyq[pl.ds(c * cr, cr), :]
            mu = jnp.mean(xv, axis=1, keepdims=True)
            xc = xv - mu
            var = jnp.mean(xc * xc, axis=1, keepdims=True)
            xhat = xc * lax.rsqrt(var + 1e-5)
            return jnp.sum(dyv * xhat, axis=0), jnp.sum(dyv, axis=0)

        def exchange(src, rbuf, send_sems, recv_sems):
            rdmas = []
            for slot, _, pid in peers:
                rdma = pltpu.make_async_remote_copy(
                    src_ref=src,
                    dst_ref=rbuf.at[slot],
                    send_sem=send_sems.at[slot],
                    recv_sem=recv_sems.at[slot],
                    device_id=pid,
                    device_id_type=pl.DeviceIdType.MESH,
                )
                rdma.start()
                rdmas.append(rdma)
            return rdmas

        dg, db = chunk_partial(0)
        for c in range(1, SPLIT):
            dgc, dbc = chunk_partial(c)
            dg = dg + dgc
            db = db + dbc
        acc_a[0, :] = dg.astype(jnp.bfloat16)
        acc_a[1, :] = db.astype(jnp.bfloat16)
        pl.semaphore_wait(barrier, N_DEV - 1)
        rdmas_a = exchange(acc_a, rbuf_a, send_a, recv_a)

        dg, db = chunk_partial(SPLIT)
        for c in range(SPLIT + 1, N_CHUNKS):
            dgc, dbc = chunk_partial(c)
            dg = dg + dgc
            db = db + dbc
        acc_b[0, :] = dg.astype(jnp.bfloat16)
        acc_b[1, :] = db.astype(jnp.bfloat16)
        rdmas_b = exchange(acc_b, rbuf_b, send_b, recv_b)

        total = acc_a[:, :].astype(jnp.float32) + acc_b[:, :].astype(jnp.float32)
        for (slot, _, _), rdma in reversed(list(zip(peers, rdmas_a))):
            rdma.wait_recv()
            total = total + rbuf_a[slot].astype(jnp.float32)
        for (slot, _, _), rdma in reversed(list(zip(peers, rdmas_b))):
            rdma.wait_recv()
            total = total + rbuf_b[slot].astype(jnp.float32)
        out_ref[:, :] = total
        for rdma in rdmas_a + rdmas_b:
            rdma.wait_send()

    return pl.pallas_call(
        body,
        out_shape=jax.ShapeDtypeStruct((2, d), jnp.float32),
        in_specs=[
            pl.BlockSpec(memory_space=pl.ANY),
            pl.BlockSpec(memory_space=pl.ANY),
        ],
        out_specs=pl.BlockSpec(memory_space=pltpu.VMEM),
        scratch_shapes=[
            pltpu.VMEM((qr, d), jnp.float32),
            pltpu.VMEM((qr, d), jnp.float32),
            pltpu.VMEM((2, d), jnp.bfloat16),
            pltpu.VMEM((2, d), jnp.bfloat16),
            pltpu.VMEM((N_DEV, 2, d), jnp.bfloat16),
            pltpu.VMEM((N_DEV, 2, d), jnp.bfloat16),
            pltpu.SemaphoreType.DMA((N_CHUNKS,)),
            pltpu.SemaphoreType.DMA((N_CHUNKS,)),
            pltpu.SemaphoreType.DMA((N_DEV,)),
            pltpu.SemaphoreType.DMA((N_DEV,)),
            pltpu.SemaphoreType.DMA((N_DEV,)),
            pltpu.SemaphoreType.DMA((N_DEV,)),
        ],
        compiler_params=pltpu.CompilerParams(collective_id=0),
    )(x, dy)
